# Initial kernel scaffold; baseline (speedup 1.0000x reference)
import jax
import jax.numpy as jnp
from jax.experimental import pallas as pl


def kernel(p1, p2, proj):
    raise NotImplementedError("write your pallas kernel here")


# speedup vs baseline: 2.3092x; 2.3092x over previous
import jax
import jax.numpy as jnp
from jax.experimental import pallas as pl
from jax.experimental.pallas import tpu as pltpu

_CHUNK = 512


def _proj_body(p1_ref, p2_ref, proj_ref, out_ref):
    proj = proj_ref[...]
    pn = proj * jax.lax.rsqrt(jnp.sum(proj * proj, axis=0, keepdims=True))
    s1 = 1.0 / (1.0 + jnp.exp(-p1_ref[...]))
    s2 = 1.0 / (1.0 + jnp.exp(-p2_ref[...]))
    z1 = jax.lax.dot(s1, pn, precision=jax.lax.Precision.HIGHEST)
    z2 = jax.lax.dot(s2, pn, precision=jax.lax.Precision.HIGHEST)
    out_ref[...] = jnp.concatenate([z1, z2], axis=1)


def _substage(xc, bk, bj, base):
    cc, _ = xc.shape
    d = 1 << bj
    i = jax.lax.broadcasted_iota(jnp.int32, (cc, 1), 0) + base
    is_lo = ((i >> bj) & 1) == 0
    partner = jnp.where(is_lo, jnp.roll(xc, -d, axis=0), jnp.roll(xc, d, axis=0))
    mn = jnp.minimum(xc, partner)
    mx = jnp.maximum(xc, partner)
    asc = ((i >> bk) & 1) == 0
    return jnp.where(asc == is_lo, mn, mx)


def _chunk_sort_body(z_ref, out_ref):
    cc = z_ref.shape[0]
    clog = cc.bit_length() - 1
    base = pl.program_id(0) * cc
    xc = z_ref[...]
    for bk in range(1, clog + 1):
        for bj in range(bk - 1, -1, -1):
            xc = _substage(xc, bk, bj, base)
    out_ref[...] = xc


def _merge_body(x_ref, out_ref):
    n, l = x_ref.shape
    m = l // 2
    cc = _CHUNK
    nch = n // cc
    nlog = n.bit_length() - 1
    clog = cc.bit_length() - 1

    for bk in range(clog + 1, nlog + 1):
        for bj in range(bk - 1, clog - 1, -1):
            dc = (1 << bj) // cc

            def cross(p, carry, bk=bk, dc=dc):
                c_lo = (p // dc) * 2 * dc + (p % dc)
                lo = x_ref[pl.ds(c_lo * cc, cc), :]
                hi = x_ref[pl.ds((c_lo + dc) * cc, cc), :]
                mn = jnp.minimum(lo, hi)
                mx = jnp.maximum(lo, hi)
                asc = ((c_lo * cc >> bk) & 1) == 0
                x_ref[pl.ds(c_lo * cc, cc), :] = jnp.where(asc, mn, mx)
                x_ref[pl.ds((c_lo + dc) * cc, cc), :] = jnp.where(asc, mx, mn)
                return carry

            jax.lax.fori_loop(0, nch // 2, cross, 0)

        def tail(c, carry, bk=bk):
            base = c * cc
            xc = x_ref[pl.ds(base, cc), :]
            for bj in range(clog - 1, -1, -1):
                xc = _substage(xc, bk, bj, base)
            x_ref[pl.ds(base, cc), :] = xc
            return carry

        jax.lax.fori_loop(0, nch, tail, 0)

    def reduce_body(c, acc):
        xc = x_ref[pl.ds(c * cc, cc), :]
        diff = xc[:, :m] - xc[:, m:]
        return acc + jnp.sum(diff * diff)

    out_ref[0, 0] = jax.lax.fori_loop(0, nch, reduce_body, jnp.float32(0.0))


def kernel(p1, p2, proj):
    n, c = p1.shape
    m = proj.shape[1]
    row_blk = 2048

    z = pl.pallas_call(
        _proj_body,
        grid=(n // row_blk,),
        in_specs=[
            pl.BlockSpec((row_blk, c), lambda i: (i, 0)),
            pl.BlockSpec((row_blk, c), lambda i: (i, 0)),
            pl.BlockSpec((c, m), lambda i: (0, 0)),
        ],
        out_specs=pl.BlockSpec((row_blk, 2 * m), lambda i: (i, 0)),
        out_shape=jax.ShapeDtypeStruct((n, 2 * m), jnp.float32),
    )(p1, p2, proj)

    zs = pl.pallas_call(
        _chunk_sort_body,
        grid=(n // _CHUNK,),
        in_specs=[pl.BlockSpec((_CHUNK, 2 * m), lambda i: (i, 0))],
        out_specs=pl.BlockSpec((_CHUNK, 2 * m), lambda i: (i, 0)),
        out_shape=jax.ShapeDtypeStruct((n, 2 * m), jnp.float32),
    )(z)

    ssq = pl.pallas_call(
        _merge_body,
        in_specs=[pl.BlockSpec((n, 2 * m), lambda: (0, 0))],
        out_specs=pl.BlockSpec(memory_space=pltpu.SMEM),
        out_shape=jax.ShapeDtypeStruct((1, 1), jnp.float32),
    )(zs)

    return ssq[0, 0] / jnp.float32(n * m)
